# baseline (speedup 1.0000x reference)
import functools

import jax
import jax.numpy as jnp
from jax import lax
from jax.experimental import pallas as pl
from jax.experimental.pallas import tpu as pltpu
from jax.experimental.pallas import tpu_sc as plsc

CUTOFF = 6.0
RESCALE = 16.0

_L = 16
_NT = 16
_NC = 2
_G = 32


def _ln_silu(h, g, b):
    mu = jnp.mean(h, axis=-1, keepdims=True)
    var = jnp.mean((h - mu) ** 2, axis=-1, keepdims=True)
    hn = (h - mu) * lax.rsqrt(var + 1e-5) * g + b
    return hn * jax.nn.sigmoid(hn)


def _edge_body(xe_ref, wig_ref, d_ref, w1_ref, b1_ref, g1_ref, be1_ref,
               w2_ref, b2_ref, g2_ref, be2_ref, w3_ref, b3_ref, out_ref):
    h = jnp.dot(xe_ref[...], w1_ref[...], preferred_element_type=jnp.float32)
    h = _ln_silu(h + b1_ref[...], g1_ref[...], be1_ref[...])
    h = jnp.dot(h, w2_ref[...], preferred_element_type=jnp.float32)
    h = _ln_silu(h + b2_ref[...], g2_ref[...], be2_ref[...])
    h = jnp.dot(h, w3_ref[...], preferred_element_type=jnp.float32) + b3_ref[...]
    d = d_ref[...] / CUTOFF
    env = jnp.where(d < 1.0,
                    1.0 - 21.0 * d**5 + 35.0 * d**6 - 15.0 * d**7,
                    0.0) / RESCALE
    wig = wig_ref[...]
    C = 128
    for m in range(9):
        acc = (wig[:, 3 * m:3 * m + 1] * h[:, 0:C]
               + wig[:, 3 * m + 1:3 * m + 2] * h[:, C:2 * C]
               + wig[:, 3 * m + 2:3 * m + 3] * h[:, 2 * C:3 * C])
        out_ref[:, C * m:C * (m + 1)] = acc * env


def _edge_embeddings(x_edge, wsl, d2, W1T, b1, g1, be1, W2T, b2, g2, be2, W3T, b3):
    E = x_edge.shape[0]
    BE = 640
    cw = lambda i: (0, 0)
    return pl.pallas_call(
        _edge_body,
        grid=(E // BE,),
        in_specs=[
            pl.BlockSpec((BE, 128), lambda i: (i, 0)),
            pl.BlockSpec((BE, 27), lambda i: (i, 0)),
            pl.BlockSpec((BE, 1), lambda i: (i, 0)),
            pl.BlockSpec((128, 128), cw),
            pl.BlockSpec((1, 128), cw),
            pl.BlockSpec((1, 128), cw),
            pl.BlockSpec((1, 128), cw),
            pl.BlockSpec((128, 128), cw),
            pl.BlockSpec((1, 128), cw),
            pl.BlockSpec((1, 128), cw),
            pl.BlockSpec((1, 128), cw),
            pl.BlockSpec((128, 384), cw),
            pl.BlockSpec((1, 384), cw),
        ],
        out_specs=pl.BlockSpec((BE, 1152), lambda i: (i, 0)),
        out_shape=jax.ShapeDtypeStruct((E, 1152), jnp.float32),
    )(x_edge, wsl, d2, W1T, b1, g1, be1, W2T, b2, g2, be2, W3T, b3)


def _make_scatter(E, NP, D, CHUNK, NCH):
    EPT = E // _NT
    SCAN_B = 2000
    NSCAN = EPT // SCAN_B
    ROWS_PT = CHUNK // _NT
    COMP = SCAN_B + _G
    NR = NCH // _NC
    mesh = plsc.VectorSubcoreMesh(core_axis_name="c", subcore_axis_name="s")

    @functools.partial(
        pl.kernel,
        mesh=mesh,
        compiler_params=pltpu.CompilerParams(needs_layout_passes=False),
        out_type=jax.ShapeDtypeStruct((NP, D), jnp.float32),
        scratch_types=[
            pltpu.VMEM((SCAN_B,), jnp.int32),
            pltpu.VMEM((COMP,), jnp.int32),
            pltpu.VMEM((COMP,), jnp.int32),
            pltpu.VMEM((_G,), jnp.int32),
            pltpu.VMEM((_G,), jnp.int32),
            pltpu.VMEM((_G, D), jnp.float32),
            pltpu.VMEM_SHARED((CHUNK + 8, D), jnp.float32),
            pltpu.SemaphoreType.DMA,
        ],
    )
    def scat(emb_hbm, dst_hbm, xp_hbm, out_hbm,
             dstb, gidb, lidb, gidx, lidx, rows, acc, sem):
        cid = lax.axis_index("c")
        sid = lax.axis_index("s")
        lane = lax.iota(jnp.int32, _L)
        zeros16 = jnp.zeros((_L,), jnp.int32)
        for r in range(NR):
            ch = _NC * r + cid
            lo = ch * CHUNK
            row0 = lo + sid * ROWS_PT
            arow = sid * ROWS_PT
            pltpu.sync_copy(xp_hbm.at[pl.ds(row0, ROWS_PT)],
                            acc.at[pl.ds(arow, ROWS_PT)])
            plsc.subcore_barrier()
            for sb in range(NSCAN):
                e0 = sid * EPT + sb * SCAN_B
                pltpu.sync_copy(dst_hbm.at[pl.ds(e0, SCAN_B)], dstb)

                def scan_body(i, off):
                    d = dstb[pl.ds(i * _L, _L)]
                    m = (d >= lo) & (d < lo + CHUNK)
                    c = plsc.cumsum(jnp.where(m, 1, 0).astype(jnp.int32))
                    pos = off + c - 1
                    eids = e0 + i * _L + lane
                    plsc.store_scatter(gidb, [pos], eids, mask=m)
                    plsc.store_scatter(lidb, [pos], d - lo, mask=m)
                    return off + plsc.all_reduce_population_count(m)

                off = lax.fori_loop(0, SCAN_B // _L, scan_body, zeros16)
                for j in range(_G // _L):
                    ppos = off + j * _L + lane
                    plsc.store_scatter(gidb, [ppos], zeros16)
                    plsc.store_scatter(lidb, [ppos],
                                       jnp.full((_L,), CHUNK, jnp.int32))
                cnt = jnp.max(off)
                ngroups = (cnt + (_G - 1)) // _G

                def drain(gi, carry):
                    for j in range(_G // _L):
                        gidx[pl.ds(j * _L, _L)] = gidb[pl.ds(gi * _G + j * _L, _L)]
                        lidx[pl.ds(j * _L, _L)] = lidb[pl.ds(gi * _G + j * _L, _L)]
                    pltpu.async_copy(emb_hbm.at[gidx], rows, sem).wait()
                    pltpu.sync_copy(rows, acc.at[lidx], add=True)
                    return carry

                lax.fori_loop(0, ngroups, drain, 0)
            plsc.subcore_barrier()
            pltpu.sync_copy(acc.at[pl.ds(arow, ROWS_PT)],
                            out_hbm.at[pl.ds(row0, ROWS_PT)])
            plsc.subcore_barrier()

    return scat


def kernel(x, x_edge, edge_distance, edge_index, wigner_and_M_mapping_inv,
           W1, b1, g1, be1, W2, b2, g2, be2, W3, b3):
    N, MALL, C = x.shape
    E = x_edge.shape[0]
    D = MALL * C
    wsl = wigner_and_M_mapping_inv[:, :, :3].reshape(E, 27)
    d2 = edge_distance.reshape(E, 1)
    dst = edge_index[1]
    emb = _edge_embeddings(
        x_edge, wsl, d2,
        W1.T, b1.reshape(1, -1), g1.reshape(1, -1), be1.reshape(1, -1),
        W2.T, b2.reshape(1, -1), g2.reshape(1, -1), be2.reshape(1, -1),
        W3.T, b3.reshape(1, -1))
    return x.at[dst].add(emb.reshape(E, MALL, C))
